# baseline (speedup 1.0000x reference)
import functools

import jax
import jax.numpy as jnp
from jax import lax
from jax.experimental import pallas as pl
from jax.experimental.pallas import tpu as pltpu
from jax.experimental.pallas import tpu_sc as plsc

_ROWS = 160000
_COLS = 256
_NC = 2
_NS = 16
_NW = _NC * _NS
_ROWS_PER = _ROWS // _NW
_CHUNK = 40
_NCHUNK = _ROWS_PER // _CHUNK
_NBUF = 8


def kernel(x_j):
    mesh = plsc.VectorSubcoreMesh(core_axis_name="c", subcore_axis_name="s")

    @functools.partial(
        pl.kernel,
        out_type=jax.ShapeDtypeStruct((_ROWS, _COLS), jnp.float32),
        mesh=mesh,
        scratch_types=[
            pltpu.VMEM((_NBUF, _CHUNK, _COLS), jnp.float32),
            pltpu.SemaphoreType.DMA((_NBUF,)),
            pltpu.SemaphoreType.DMA((_NBUF,)),
        ],
    )
    def sc_copy(x_hbm, out_hbm, bufs, in_sems, out_sems):
        wid = lax.axis_index("s") * _NC + lax.axis_index("c")
        base = wid * _ROWS_PER

        def in_copy(i):
            return pltpu.make_async_copy(
                x_hbm.at[pl.ds(base + i * _CHUNK, _CHUNK), :],
                bufs.at[i % _NBUF],
                in_sems.at[i % _NBUF],
            )

        def out_copy(i):
            return pltpu.make_async_copy(
                bufs.at[i % _NBUF],
                out_hbm.at[pl.ds(base + i * _CHUNK, _CHUNK), :],
                out_sems.at[i % _NBUF],
            )

        for i in range(4):
            in_copy(i).start()
        for i in range(_NCHUNK):
            in_copy(i).wait()
            out_copy(i).start()
            if i >= 4:
                out_copy(i - 4).wait()
            if i + 4 < _NCHUNK:
                in_copy(i + 4).start()
        for i in range(_NCHUNK - 4, _NCHUNK):
            out_copy(i).wait()

    return sc_copy(x_j)


# speedup vs baseline: 1.3239x; 1.3239x over previous
import jax
import jax.numpy as jnp
from jax.experimental import pallas as pl
from jax.experimental.pallas import tpu as pltpu

_ROWS = 160000
_COLS = 256
_BLOCK = 10000


def _copy_body(x_ref, o_ref):
    o_ref[...] = x_ref[...]


def kernel(x_j):
    grid = (_ROWS // _BLOCK,)
    return pl.pallas_call(
        _copy_body,
        grid=grid,
        in_specs=[pl.BlockSpec((_BLOCK, _COLS), lambda i: (i, 0))],
        out_specs=pl.BlockSpec((_BLOCK, _COLS), lambda i: (i, 0)),
        out_shape=jax.ShapeDtypeStruct(x_j.shape, x_j.dtype),
        compiler_params=pltpu.CompilerParams(
            dimension_semantics=("parallel",),
        ),
    )(x_j)


# speedup vs baseline: 1.3400x; 1.0122x over previous
import jax
import jax.numpy as jnp
from jax.experimental import pallas as pl
from jax.experimental.pallas import tpu as pltpu

_ROWS = 160000
_COLS = 256
_BLOCK = 12800


def _copy_body(x_ref, o_ref):
    o_ref[...] = x_ref[...]


def kernel(x_j):
    grid = (pl.cdiv(_ROWS, _BLOCK),)
    return pl.pallas_call(
        _copy_body,
        grid=grid,
        in_specs=[pl.BlockSpec((_BLOCK, _COLS), lambda i: (i, 0))],
        out_specs=pl.BlockSpec((_BLOCK, _COLS), lambda i: (i, 0)),
        out_shape=jax.ShapeDtypeStruct(x_j.shape, x_j.dtype),
        compiler_params=pltpu.CompilerParams(
            dimension_semantics=("parallel",),
        ),
    )(x_j)


# speedup vs baseline: 1.3415x; 1.0011x over previous
import jax
import jax.numpy as jnp
from jax.experimental import pallas as pl
from jax.experimental.pallas import tpu as pltpu

_ROWS = 160000
_COLS = 256
_BLOCK = 14000


def _copy_body(x_ref, o_ref):
    o_ref[...] = x_ref[...]


def kernel(x_j):
    grid = (pl.cdiv(_ROWS, _BLOCK),)
    return pl.pallas_call(
        _copy_body,
        grid=grid,
        in_specs=[pl.BlockSpec((_BLOCK, _COLS), lambda i: (i, 0))],
        out_specs=pl.BlockSpec((_BLOCK, _COLS), lambda i: (i, 0)),
        out_shape=jax.ShapeDtypeStruct(x_j.shape, x_j.dtype),
        compiler_params=pltpu.CompilerParams(
            dimension_semantics=("parallel",),
        ),
    )(x_j)
